# baseline (speedup 1.0000x reference)
import functools

import jax
import jax.numpy as jnp
from jax import lax
from jax.experimental import pallas as pl
from jax.experimental.pallas import tpu as pltpu
from jax.experimental.pallas import tpu_sc as plsc

C = 128
DEG = 32
NW = 32
TILE_NODES = 320
SUB = 128
NSUB = 3
TILE_EXT = SUB * NSUB
N_PAD = NW * TILE_NODES


def _matmul_kernel(x_ref, w_ref, o_ref):
  o_ref[...] = jnp.dot(x_ref[...], w_ref[...], preferred_element_type=jnp.float32)


def _gru_kernel(s_ref, x_ref, wih_ref, whh_ref, bih_ref, bhh_ref, o_ref):
  s = s_ref[...]
  h = x_ref[...]
  dn = (((1,), (1,)), ((), ()))
  gi = lax.dot_general(s, wih_ref[...], dn, preferred_element_type=jnp.float32)
  gi = gi + bih_ref[...]
  gh = lax.dot_general(h, whh_ref[...], dn, preferred_element_type=jnp.float32)
  gh = gh + bhh_ref[...]
  r = jax.nn.sigmoid(gi[:, :C] + gh[:, :C])
  z = jax.nn.sigmoid(gi[:, C:2 * C] + gh[:, C:2 * C])
  n = jnp.tanh(gi[:, 2 * C:] + r * gh[:, 2 * C:])
  o_ref[...] = (1.0 - z) * n + z * h


def _gather_sum_body(table_hbm, sidx_hbm, out_hbm, idx_v, acc_v, sem):
  c = lax.axis_index("c")
  s = lax.axis_index("s")
  wid = s * 2 + c
  pltpu.sync_copy(sidx_hbm.at[wid], idx_v)
  for sub in range(NSUB):
    dst = acc_v.at[pl.ds(sub * SUB, SUB)]
    pltpu.async_copy(table_hbm.at[idx_v.at[sub * DEG]], dst, sem).wait()

    def body(d, carry):
      pltpu.async_copy(table_hbm.at[idx_v.at[sub * DEG + d]], dst, sem,
                       add=True).wait()
      return carry

    lax.fori_loop(1, DEG, body, 0)
  pltpu.sync_copy(acc_v.at[pl.ds(0, TILE_NODES)],
                  out_hbm.at[pl.ds(wid * TILE_NODES, TILE_NODES)])


def _make_gather_sum():
  mesh = plsc.VectorSubcoreMesh(core_axis_name="c", subcore_axis_name="s")
  return pl.kernel(
      _gather_sum_body,
      out_type=jax.ShapeDtypeStruct((N_PAD, C), jnp.float32),
      mesh=mesh,
      scratch_types=[
          pltpu.VMEM((NSUB * DEG, SUB), jnp.int32),
          pltpu.VMEM((TILE_EXT, C), jnp.float32),
          pltpu.SemaphoreType.DMA,
      ],
  )


@jax.jit
def kernel(x, edge_index, weight, W_ih, W_hh, b_ih, b_hh):
  n = x.shape[0]
  x_pad = jnp.zeros((N_PAD, C), jnp.float32).at[:n].set(x)
  e = edge_index.astype(jnp.int32)
  e_pad = jnp.full((N_PAD, DEG), n, jnp.int32).at[:n].set(e)
  e_t = e_pad.reshape(NW, TILE_NODES, DEG)
  e_t = jnp.concatenate(
      [e_t, jnp.full((NW, TILE_EXT - TILE_NODES, DEG), n, jnp.int32)], axis=1)
  sidx = e_t.reshape(NW, NSUB, SUB, DEG).transpose(0, 1, 3, 2).reshape(
      NW, NSUB * DEG, SUB)

  bm = 512
  m_pad = pl.pallas_call(
      _matmul_kernel,
      grid=(N_PAD // bm,),
      in_specs=[
          pl.BlockSpec((bm, C), lambda i: (i, 0)),
          pl.BlockSpec((C, C), lambda i: (0, 0)),
      ],
      out_specs=pl.BlockSpec((bm, C), lambda i: (i, 0)),
      out_shape=jax.ShapeDtypeStruct((N_PAD, C), jnp.float32),
  )(x_pad, weight[0])

  s_pad = _make_gather_sum()(m_pad, sidx)

  out = pl.pallas_call(
      _gru_kernel,
      grid=(N_PAD // bm,),
      in_specs=[
          pl.BlockSpec((bm, C), lambda i: (i, 0)),
          pl.BlockSpec((bm, C), lambda i: (i, 0)),
          pl.BlockSpec((3 * C, C), lambda i: (0, 0)),
          pl.BlockSpec((3 * C, C), lambda i: (0, 0)),
          pl.BlockSpec((1, 3 * C), lambda i: (0, 0)),
          pl.BlockSpec((1, 3 * C), lambda i: (0, 0)),
      ],
      out_specs=pl.BlockSpec((bm, C), lambda i: (i, 0)),
      out_shape=jax.ShapeDtypeStruct((N_PAD, C), jnp.float32),
  )(s_pad, x_pad, W_ih, W_hh, b_ih.reshape(1, 3 * C), b_hh.reshape(1, 3 * C))

  return out[:n]
